# baseline (speedup 1.0000x reference)
import functools

import jax
import jax.numpy as jnp
from jax import lax
from jax.experimental import pallas as pl
from jax.experimental.pallas import tpu as pltpu
from jax.experimental.pallas import tpu_sc as plsc

_SEQ = 200
_DIM = 64
_BATCH = 4096
_REP_S = 32
_REP_V = 4


@jax.jit
def _pos_broadcast(pos_embedding):
    info = plsc.get_sparse_core_info()
    nw = info.num_cores * info.num_subcores
    per_w = _BATCH // nw
    half = per_w // 2
    n_dma_s = half // _REP_S
    n_dma_v = half // _REP_V
    fill_per_sub = _REP_S // info.num_subcores

    mesh = plsc.VectorSubcoreMesh(core_axis_name="c", subcore_axis_name="s")

    @functools.partial(
        pl.kernel,
        mesh=mesh,
        out_type=jax.ShapeDtypeStruct((_BATCH, _SEQ, _DIM), jnp.float32),
        scratch_types=[
            pltpu.VMEM_SHARED((_REP_S, _SEQ, _DIM), jnp.float32),
            pltpu.VMEM((_REP_V, _SEQ, _DIM), jnp.float32),
            pltpu.SemaphoreType.DMA,
            pltpu.SemaphoreType.DMA,
        ],
    )
    def k(table_hbm, out_hbm, rep_s, rep_v, sem_s, sem_v):
        sid = lax.axis_index("s")
        for r in range(_REP_V):
            pltpu.sync_copy(table_hbm, rep_v.at[r])
        for r in range(fill_per_sub):
            pltpu.sync_copy(table_hbm, rep_s.at[sid * fill_per_sub + r])
        plsc.subcore_barrier()
        wid = sid * info.num_cores + lax.axis_index("c")
        base = wid * per_w
        copies = [
            pltpu.async_copy(
                rep_s, out_hbm.at[pl.ds(base + i * _REP_S, _REP_S)], sem_s
            )
            for i in range(n_dma_s)
        ] + [
            pltpu.async_copy(
                rep_v,
                out_hbm.at[pl.ds(base + half + i * _REP_V, _REP_V)],
                sem_v,
            )
            for i in range(n_dma_v)
        ]
        for c in copies:
            c.wait()

    out = k(pos_embedding)
    return lax.bitcast_convert_type(out, jnp.float32)


def kernel(input_ids, pos_embedding):
    del input_ids
    return _pos_broadcast(pos_embedding)


# speedup vs baseline: 1.0204x; 1.0204x over previous
import functools

import jax
import jax.numpy as jnp
from jax import lax
from jax.experimental import pallas as pl
from jax.experimental.pallas import tpu as pltpu
from jax.experimental.pallas import tpu_sc as plsc

_SEQ = 200
_DIM = 64
_BATCH = 4096
_REP = 4


@jax.jit
def _pos_broadcast(pos_embedding):
    info = plsc.get_sparse_core_info()
    nw = info.num_cores * info.num_subcores
    per_w = _BATCH // nw
    n_dma = per_w // _REP

    mesh = plsc.VectorSubcoreMesh(core_axis_name="c", subcore_axis_name="s")

    @functools.partial(
        pl.kernel,
        mesh=mesh,
        out_type=jax.ShapeDtypeStruct((_BATCH, _SEQ, _DIM), jnp.float32),
        scratch_types=[
            pltpu.VMEM((_REP, _SEQ, _DIM), jnp.float32),
            pltpu.SemaphoreType.DMA,
        ],
    )
    def k(table_hbm, out_hbm, rep_v, sem):
        for r in range(_REP):
            pltpu.sync_copy(table_hbm, rep_v.at[r])
        wid = lax.axis_index("s") * info.num_cores + lax.axis_index("c")
        base = wid * per_w
        copies = [
            pltpu.async_copy(
                rep_v, out_hbm.at[pl.ds(base + i * _REP, _REP)], sem
            )
            for i in range(n_dma)
        ]
        for c in copies:
            c.wait()

    return k(pos_embedding)


def kernel(input_ids, pos_embedding):
    del input_ids
    return _pos_broadcast(pos_embedding)
